# baseline (device time: 24947 ns/iter reference)
import contextlib
import os

import jax
import jax.numpy as jnp
from jax import lax
from jax.experimental import pallas as pl
from jax.experimental.pallas import tpu as pltpu

N_DEV = 4
N_LAYERS = 3

_SCOPES = bool(os.environ.get("KERNEL_SCOPES"))


def _scope(name):
    return jax.named_scope(name) if _SCOPES else contextlib.nullcontext()


def kernel(x, Win0, Wout0, Win1, Wout1, Win2, Wout2):
    b, d_shard = x.shape
    _, h_dim = Win0.shape

    def body(x_ref, win0_ref, wout0_ref, win1_ref, wout1_ref, win2_ref,
             wout2_ref, out_ref, src_ref, comm_ref, send_sems, recv_sems):
        my = lax.axis_index("i")
        pA = my ^ 1
        pB = 3 - my

        with _scope("barrier"):
            barrier_sem = pltpu.get_barrier_semaphore()
            for p in (pA, pB):
                pl.semaphore_signal(
                    barrier_sem, inc=1,
                    device_id=(p,),
                    device_id_type=pl.DeviceIdType.MESH,
                )
            pl.semaphore_wait(barrier_sem, 2)

        wins = [win0_ref, win1_ref, win2_ref]
        wouts = [wout0_ref, wout1_ref, wout2_ref]

        xv = x_ref[:, :].astype(jnp.bfloat16)
        all_rdmas = []
        for l in range(N_LAYERS):
            with _scope(f"mm1#l={l}"):
                partial = jnp.dot(xv, wins[l][:, :].astype(jnp.bfloat16),
                                  preferred_element_type=jnp.float32)
                src_ref[l, 0, :, :] = partial.astype(jnp.bfloat16)

            with _scope(f"phaseA#l={l}"):
                rdmaA = pltpu.make_async_remote_copy(
                    src_ref=src_ref.at[l, 0],
                    dst_ref=comm_ref.at[l, 0],
                    send_sem=send_sems.at[l, 0],
                    recv_sem=recv_sems.at[l, 0],
                    device_id=(pA,),
                    device_id_type=pl.DeviceIdType.MESH,
                )
                rdmaA.start()
                all_rdmas.append(rdmaA)
                rdmaA.wait_recv()
                sum2 = partial + comm_ref[l, 0, :, :].astype(jnp.float32)

            with _scope(f"phaseB#l={l}"):
                src_ref[l, 1, :, :] = sum2.astype(jnp.bfloat16)
                rdmaB = pltpu.make_async_remote_copy(
                    src_ref=src_ref.at[l, 1],
                    dst_ref=comm_ref.at[l, 1],
                    send_sem=send_sems.at[l, 1],
                    recv_sem=recv_sems.at[l, 1],
                    device_id=(pB,),
                    device_id_type=pl.DeviceIdType.MESH,
                )
                rdmaB.start()
                all_rdmas.append(rdmaB)
                rdmaB.wait_recv()
                acc = sum2 + comm_ref[l, 1, :, :].astype(jnp.float32)

            with _scope(f"mm2#l={l}"):
                h = jnp.maximum(acc, 0.0).astype(jnp.bfloat16)
                xv = jnp.dot(h, wouts[l][:, :].astype(jnp.bfloat16),
                             preferred_element_type=jnp.float32
                             ).astype(jnp.bfloat16)

        out_ref[:, :] = xv.astype(jnp.float32)
        for rdma in all_rdmas:
            rdma.wait_send()

    return pl.pallas_call(
        body,
        out_shape=jax.ShapeDtypeStruct((b, d_shard), jnp.float32),
        in_specs=[pl.BlockSpec(memory_space=pltpu.VMEM)] * 7,
        out_specs=pl.BlockSpec(memory_space=pltpu.VMEM),
        scratch_shapes=[
            pltpu.VMEM((N_LAYERS, 2, b, h_dim), jnp.bfloat16),
            pltpu.VMEM((N_LAYERS, 2, b, h_dim), jnp.bfloat16),
            pltpu.SemaphoreType.DMA((N_LAYERS, 2)),
            pltpu.SemaphoreType.DMA((N_LAYERS, 2)),
        ],
        compiler_params=pltpu.CompilerParams(collective_id=0),
    )(x, Win0, Wout0, Win1, Wout1, Win2, Wout2)


# device time: 20651 ns/iter; 1.2080x vs baseline; 1.2080x over previous
import contextlib
import os

import jax
import jax.numpy as jnp
from jax import lax
from jax.experimental import pallas as pl
from jax.experimental.pallas import tpu as pltpu

N_DEV = 4
N_LAYERS = 3

_SCOPES = bool(os.environ.get("KERNEL_SCOPES"))


def _scope(name):
    return jax.named_scope(name) if _SCOPES else contextlib.nullcontext()


def kernel(x, Win0, Wout0, Win1, Wout1, Win2, Wout2):
    b, d_shard = x.shape
    _, h_dim = Win0.shape

    def body(x_ref, win0_ref, wout0_ref, win1_ref, wout1_ref, win2_ref,
             wout2_ref, out_ref, src_ref, comm_ref, send_sems, recv_sems):
        my = lax.axis_index("i")

        with _scope("barrier"):
            barrier_sem = pltpu.get_barrier_semaphore()
            for d in range(1, N_DEV):
                pl.semaphore_signal(
                    barrier_sem, inc=1,
                    device_id=((my + d) % N_DEV,),
                    device_id_type=pl.DeviceIdType.MESH,
                )
            pl.semaphore_wait(barrier_sem, N_DEV - 1)

        wins = [win0_ref, win1_ref, win2_ref]
        wouts = [wout0_ref, wout1_ref, wout2_ref]

        xv = x_ref[:, :].astype(jnp.bfloat16)
        all_rdmas = []
        for l in range(N_LAYERS):
            with _scope(f"mm1#l={l}"):
                partial = jnp.dot(xv, wins[l][:, :].astype(jnp.bfloat16),
                                  preferred_element_type=jnp.float32)
                src_ref[l, :, :] = partial.astype(jnp.bfloat16)

            with _scope(f"issue#l={l}"):
                rdmas = {}
                for d in (2, 1, 3):
                    rdma = pltpu.make_async_remote_copy(
                        src_ref=src_ref.at[l],
                        dst_ref=comm_ref.at[l, d - 1],
                        send_sem=send_sems.at[l, d - 1],
                        recv_sem=recv_sems.at[l, d - 1],
                        device_id=((my + d) % N_DEV,),
                        device_id_type=pl.DeviceIdType.MESH,
                    )
                    rdma.start()
                    rdmas[d] = rdma
                    all_rdmas.append(rdma)

            acc = partial
            for d in (1, 3, 2):
                with _scope(f"waitrecv#l={l}#d={d}"):
                    rdmas[d].wait_recv()
                with _scope(f"add#l={l}#d={d}"):
                    acc = acc + comm_ref[l, d - 1, :, :].astype(jnp.float32)
            with _scope(f"mm2#l={l}"):
                h = jnp.maximum(acc, 0.0).astype(jnp.bfloat16)
                xv = jnp.dot(h, wouts[l][:, :].astype(jnp.bfloat16),
                             preferred_element_type=jnp.float32
                             ).astype(jnp.bfloat16)

        out_ref[:, :] = xv.astype(jnp.float32)
        for rdma in all_rdmas:
            rdma.wait_send()

    return pl.pallas_call(
        body,
        out_shape=jax.ShapeDtypeStruct((b, d_shard), jnp.float32),
        in_specs=[pl.BlockSpec(memory_space=pltpu.VMEM)] * 7,
        out_specs=pl.BlockSpec(memory_space=pltpu.VMEM),
        scratch_shapes=[
            pltpu.VMEM((N_LAYERS, b, h_dim), jnp.bfloat16),
            pltpu.VMEM((N_LAYERS, N_DEV - 1, b, h_dim), jnp.bfloat16),
            pltpu.SemaphoreType.DMA((N_LAYERS, N_DEV - 1)),
            pltpu.SemaphoreType.DMA((N_LAYERS, N_DEV - 1)),
        ],
        compiler_params=pltpu.CompilerParams(collective_id=0),
    )(x, Win0, Wout0, Win1, Wout1, Win2, Wout2)
